# baseline (device time: 1736366 ns/iter reference)
import jax
import jax.numpy as jnp
from jax import lax
from jax.experimental import pallas as pl
from jax.experimental.pallas import tpu as pltpu

NDEV = 32
M, N = 4096, 8192
NH = N // 2
CH = M // NDEV


def _silu_f32(v_bf16):
    y = v_bf16.astype(jnp.float32)
    return y * (1.0 / (1.0 + jnp.exp(-y)))


def _ring_ar_silu(partial):
    def body(p_ref, o_ref, commR, commL, accR, accL, lbufR, lbufL,
             obufR, obufL, sendR, recvR, sendL, recvL,
             lsemR, lsemL, osemR, osemL, creditR, creditL):
        my = lax.axis_index("i")
        left = lax.rem(my + NDEV - 1, NDEV)
        right = lax.rem(my + 1, NDEV)

        barrier = pltpu.get_barrier_semaphore()
        for nbr in (left, right):
            pl.semaphore_signal(barrier, inc=1, device_id=(nbr,),
                                device_id_type=pl.DeviceIdType.MESH)
        pl.semaphore_wait(barrier, 2)

        cpR = pltpu.make_async_copy(
            p_ref.at[pl.ds(my * CH, CH), pl.ds(0, NH)], accR, lsemR)
        cpL = pltpu.make_async_copy(
            p_ref.at[pl.ds(my * CH, CH), pl.ds(NH, NH)], accL, lsemL)
        cpR.start()
        cpL.start()
        cpR.wait()
        cpL.wait()

        def rdma_to(src, dst, send_sem, recv_sem, target):
            return pltpu.make_async_remote_copy(
                src_ref=src, dst_ref=dst, send_sem=send_sem,
                recv_sem=recv_sem, device_id=(target,),
                device_id_type=pl.DeviceIdType.MESH)

        pl.semaphore_signal(creditR, inc=2)
        pl.semaphore_signal(creditL, inc=2)

        def wait_credits():
            pl.semaphore_wait(creditR, 1)
            pl.semaphore_wait(creditL, 1)

        def free_slots():
            pl.semaphore_signal(creditR, inc=1, device_id=(left,),
                                device_id_type=pl.DeviceIdType.MESH)
            pl.semaphore_signal(creditL, inc=1, device_id=(right,),
                                device_id_type=pl.DeviceIdType.MESH)

        def rs_step(s, _):
            slot = lax.rem(s, 2)
            cR = lax.rem(my - (s + 1) + 2 * NDEV, NDEV)
            cL = lax.rem(my + (s + 1), NDEV)
            lcpR = pltpu.make_async_copy(
                p_ref.at[pl.ds(cR * CH, CH), pl.ds(0, NH)], lbufR, lsemR)
            lcpL = pltpu.make_async_copy(
                p_ref.at[pl.ds(cL * CH, CH), pl.ds(NH, NH)], lbufL, lsemL)
            lcpR.start()
            lcpL.start()
            wait_credits()
            rR = rdma_to(accR, commR.at[slot], sendR.at[slot],
                         recvR.at[slot], right)
            rL = rdma_to(accL, commL.at[slot], sendL.at[slot],
                         recvL.at[slot], left)
            rR.start()
            rL.start()
            rR.wait()
            lcpR.wait()
            accR[...] = commR[slot] + lbufR[...]
            rL.wait()
            lcpL.wait()
            accL[...] = commL[slot] + lbufL[...]
            free_slots()
            return 0

        lax.fori_loop(0, NDEV - 1, rs_step, 0)


        def store(chunk_idx, vals_bf16, col_off, obuf, osem):
            obuf[...] = _silu_f32(vals_bf16)
            ocp = pltpu.make_async_copy(
                obuf, o_ref.at[pl.ds(chunk_idx * CH, CH), pl.ds(col_off, NH)],
                osem)
            ocp.start()
            ocp.wait()

        slot0 = (NDEV - 1) % 2
        wait_credits()
        rR = rdma_to(accR, commR.at[slot0], sendR.at[slot0],
                     recvR.at[slot0], right)
        rL = rdma_to(accL, commL.at[slot0], sendL.at[slot0],
                     recvL.at[slot0], left)
        rR.start()
        rL.start()
        store(lax.rem(my + 1, NDEV), accR[...], 0, obufR, osemR)
        store(lax.rem(my + NDEV - 1, NDEV), accL[...], NH, obufL, osemL)
        rR.wait()
        store(my, commR[slot0], 0, obufR, osemR)
        rL.wait()
        store(my, commL[slot0], NH, obufL, osemL)

        def ag_step(t, _):
            u = (NDEV - 1) + t
            slot = lax.rem(u, 2)
            prev = lax.rem(u + 1, 2)
            wait_credits()
            rR = rdma_to(commR.at[prev], commR.at[slot], sendR.at[slot],
                         recvR.at[slot], right)
            rL = rdma_to(commL.at[prev], commL.at[slot], sendL.at[slot],
                         recvL.at[slot], left)
            rR.start()
            rL.start()
            rR.wait()
            rL.wait()
            free_slots()
            store(lax.rem(my - t + 2 * NDEV, NDEV), commR[slot], 0,
                  obufR, osemR)
            store(lax.rem(my + t, NDEV), commL[slot], NH, obufL, osemL)
            return 0

        lax.fori_loop(1, NDEV - 1, ag_step, 0)

        wait_credits()

    return pl.pallas_call(
        body,
        out_shape=jax.ShapeDtypeStruct((M, N), jnp.float32),
        in_specs=[pl.BlockSpec(memory_space=pltpu.MemorySpace.HBM)],
        out_specs=pl.BlockSpec(memory_space=pltpu.MemorySpace.HBM),
        scratch_shapes=[
            pltpu.VMEM((2, CH, NH), jnp.bfloat16),
            pltpu.VMEM((2, CH, NH), jnp.bfloat16),
            pltpu.VMEM((CH, NH), jnp.bfloat16),
            pltpu.VMEM((CH, NH), jnp.bfloat16),
            pltpu.VMEM((CH, NH), jnp.bfloat16),
            pltpu.VMEM((CH, NH), jnp.bfloat16),
            pltpu.VMEM((CH, NH), jnp.float32),
            pltpu.VMEM((CH, NH), jnp.float32),
            pltpu.SemaphoreType.DMA((2,)),
            pltpu.SemaphoreType.DMA((2,)),
            pltpu.SemaphoreType.DMA((2,)),
            pltpu.SemaphoreType.DMA((2,)),
            pltpu.SemaphoreType.DMA,
            pltpu.SemaphoreType.DMA,
            pltpu.SemaphoreType.DMA,
            pltpu.SemaphoreType.DMA,
            pltpu.SemaphoreType.REGULAR,
            pltpu.SemaphoreType.REGULAR,
        ],
        compiler_params=pltpu.CompilerParams(collective_id=0),
    )(partial)


def kernel(x, w_mat):
    partial = jnp.dot(x, w_mat, preferred_element_type=jnp.float32)
    return _ring_ar_silu(partial.astype(jnp.bfloat16))


# device time: 1054603 ns/iter; 1.6465x vs baseline; 1.6465x over previous
import jax
import jax.numpy as jnp
from jax import lax
from jax.experimental import pallas as pl
from jax.experimental.pallas import tpu as pltpu

NDEV = 32
M, N = 4096, 8192
NH = N // 2
CH = M // NDEV

_PLANE = [(0, 0), (1, 0), (1, 1), (0, 1), (0, 2), (1, 2), (1, 3), (0, 3)]
_LOGICAL = {}
for _z in range(4):
    for _p, (_x, _y) in enumerate(_PLANE):
        _LOGICAL[(_x, _y, _z)] = _z * 8 + _p

_HAM = []
for _z in range(4):
    _ys = range(4) if _z % 2 == 0 else range(3, -1, -1)
    _HAM += [(0, _y, _z) for _y in _ys]
for _z in range(3, -1, -1):
    _ys = range(4) if _z % 2 == 1 else range(3, -1, -1)
    _HAM += [(1, _y, _z) for _y in _ys]
assert len(set(_HAM)) == NDEV
for _a, _b in zip(_HAM, _HAM[1:] + _HAM[:1]):
    assert sum(abs(i - j) for i, j in zip(_a, _b)) == 1, (_a, _b)

_PERM = [_LOGICAL[c] for c in _HAM]
_RINGPOS = [0] * NDEV
for _r, _l in enumerate(_PERM):
    _RINGPOS[_l] = _r
_RIGHT = [_PERM[(_RINGPOS[l] + 1) % NDEV] for l in range(NDEV)]
_LEFT = [_PERM[(_RINGPOS[l] - 1) % NDEV] for l in range(NDEV)]


def _silu_f32(v_bf16):
    y = v_bf16.astype(jnp.float32)
    return y * (1.0 / (1.0 + jnp.exp(-y)))


def _ring_ar_silu(partial, ids):
    def body(ids_ref, p_ref, o_ref, commR, commL, accR, accL, lbufR, lbufL,
             obufR, obufL, sendR, recvR, sendL, recvL,
             lsemR, lsemL, osemR, osemL, creditR, creditL):
        r = ids_ref[0]
        right = ids_ref[1]
        left = ids_ref[2]

        barrier = pltpu.get_barrier_semaphore()
        for nbr in (left, right):
            pl.semaphore_signal(barrier, inc=1, device_id=(nbr,),
                                device_id_type=pl.DeviceIdType.MESH)
        pl.semaphore_wait(barrier, 2)

        cpR = pltpu.make_async_copy(
            p_ref.at[pl.ds(r * CH, CH), pl.ds(0, NH)], accR, lsemR)
        cpL = pltpu.make_async_copy(
            p_ref.at[pl.ds(r * CH, CH), pl.ds(NH, NH)], accL, lsemL)
        cpR.start()
        cpL.start()
        cpR.wait()
        cpL.wait()

        def rdma_to(src, dst, send_sem, recv_sem, target):
            return pltpu.make_async_remote_copy(
                src_ref=src, dst_ref=dst, send_sem=send_sem,
                recv_sem=recv_sem, device_id=(target,),
                device_id_type=pl.DeviceIdType.MESH)

        pl.semaphore_signal(creditR, inc=2)
        pl.semaphore_signal(creditL, inc=2)

        def wait_credits():
            pl.semaphore_wait(creditR, 1)
            pl.semaphore_wait(creditL, 1)

        def free_slots():
            pl.semaphore_signal(creditR, inc=1, device_id=(left,),
                                device_id_type=pl.DeviceIdType.MESH)
            pl.semaphore_signal(creditL, inc=1, device_id=(right,),
                                device_id_type=pl.DeviceIdType.MESH)

        def rs_step(s, _):
            slot = lax.rem(s, 2)
            cR = lax.rem(r - (s + 1) + 2 * NDEV, NDEV)
            cL = lax.rem(r + (s + 1), NDEV)
            lcpR = pltpu.make_async_copy(
                p_ref.at[pl.ds(cR * CH, CH), pl.ds(0, NH)], lbufR, lsemR)
            lcpL = pltpu.make_async_copy(
                p_ref.at[pl.ds(cL * CH, CH), pl.ds(NH, NH)], lbufL, lsemL)
            lcpR.start()
            lcpL.start()
            wait_credits()
            rR = rdma_to(accR, commR.at[slot], sendR.at[slot],
                         recvR.at[slot], right)
            rL = rdma_to(accL, commL.at[slot], sendL.at[slot],
                         recvL.at[slot], left)
            rR.start()
            rL.start()
            rR.wait()
            lcpR.wait()
            accR[...] = commR[slot] + lbufR[...]
            rL.wait()
            lcpL.wait()
            accL[...] = commL[slot] + lbufL[...]
            free_slots()
            return 0

        lax.fori_loop(0, NDEV - 1, rs_step, 0)


        def store(chunk_idx, vals_bf16, col_off, obuf, osem):
            obuf[...] = _silu_f32(vals_bf16)
            ocp = pltpu.make_async_copy(
                obuf, o_ref.at[pl.ds(chunk_idx * CH, CH), pl.ds(col_off, NH)],
                osem)
            ocp.start()
            ocp.wait()

        slot0 = (NDEV - 1) % 2
        wait_credits()
        rR = rdma_to(accR, commR.at[slot0], sendR.at[slot0],
                     recvR.at[slot0], right)
        rL = rdma_to(accL, commL.at[slot0], sendL.at[slot0],
                     recvL.at[slot0], left)
        rR.start()
        rL.start()
        store(lax.rem(r + 1, NDEV), accR[...], 0, obufR, osemR)
        store(lax.rem(r + NDEV - 1, NDEV), accL[...], NH, obufL, osemL)
        rR.wait()
        store(r, commR[slot0], 0, obufR, osemR)
        rL.wait()
        store(r, commL[slot0], NH, obufL, osemL)

        def ag_step(t, _):
            u = (NDEV - 1) + t
            slot = lax.rem(u, 2)
            prev = lax.rem(u + 1, 2)
            wait_credits()
            rR = rdma_to(commR.at[prev], commR.at[slot], sendR.at[slot],
                         recvR.at[slot], right)
            rL = rdma_to(commL.at[prev], commL.at[slot], sendL.at[slot],
                         recvL.at[slot], left)
            rR.start()
            rL.start()
            rR.wait()
            rL.wait()
            free_slots()
            store(lax.rem(r - t + 2 * NDEV, NDEV), commR[slot], 0,
                  obufR, osemR)
            store(lax.rem(r + t, NDEV), commL[slot], NH, obufL, osemL)
            return 0

        lax.fori_loop(1, NDEV - 1, ag_step, 0)

        wait_credits()

    return pl.pallas_call(
        body,
        out_shape=jax.ShapeDtypeStruct((M, N), jnp.float32),
        in_specs=[
            pl.BlockSpec(memory_space=pltpu.MemorySpace.SMEM),
            pl.BlockSpec(memory_space=pltpu.MemorySpace.HBM),
        ],
        out_specs=pl.BlockSpec(memory_space=pltpu.MemorySpace.HBM),
        scratch_shapes=[
            pltpu.VMEM((2, CH, NH), jnp.bfloat16),
            pltpu.VMEM((2, CH, NH), jnp.bfloat16),
            pltpu.VMEM((CH, NH), jnp.bfloat16),
            pltpu.VMEM((CH, NH), jnp.bfloat16),
            pltpu.VMEM((CH, NH), jnp.bfloat16),
            pltpu.VMEM((CH, NH), jnp.bfloat16),
            pltpu.VMEM((CH, NH), jnp.float32),
            pltpu.VMEM((CH, NH), jnp.float32),
            pltpu.SemaphoreType.DMA((2,)),
            pltpu.SemaphoreType.DMA((2,)),
            pltpu.SemaphoreType.DMA((2,)),
            pltpu.SemaphoreType.DMA((2,)),
            pltpu.SemaphoreType.DMA,
            pltpu.SemaphoreType.DMA,
            pltpu.SemaphoreType.DMA,
            pltpu.SemaphoreType.DMA,
            pltpu.SemaphoreType.REGULAR,
            pltpu.SemaphoreType.REGULAR,
        ],
        compiler_params=pltpu.CompilerParams(collective_id=0),
    )(ids, partial)


def kernel(x, w_mat):
    partial = jnp.dot(x, w_mat, preferred_element_type=jnp.float32)
    my = lax.axis_index("i")
    ids = jnp.stack([
        jnp.asarray(_RINGPOS, jnp.int32)[my],
        jnp.asarray(_RIGHT, jnp.int32)[my],
        jnp.asarray(_LEFT, jnp.int32)[my],
    ])
    return _ring_ar_silu(partial.astype(jnp.bfloat16), ids)
